# baseline (speedup 1.0000x reference)
import jax
import jax.numpy as jnp
from jax import lax
from jax.experimental import pallas as pl
from jax.experimental.pallas import tpu as pltpu
from jax.experimental.pallas import tpu_sc as plsc

_NW = 32
_N = 16384
_RPW = _N // _NW
_UNROLL = 4
_ITERS = _RPW // (16 * _UNROLL)


def _splat_i(v):
    return jnp.full((16,), v, dtype=jnp.int32)


def _rsplat(vec, k):
    return jnp.take_along_axis(vec, _splat_i(k), axis=0)


def _round_bf16(v):
    u = plsc.bitcast(v, jnp.int32)
    u = (u + jnp.int32(0x8000)) & jnp.int32(-0x10000)
    return plsc.bitcast(u, jnp.float32)


def _sc_body(x_hbm, w1_hbm, b1_hbm, w2_hbm, b2_hbm, val_hbm, idx_hbm,
             xv, w1v, b1v, w2v, b2v, w1sp, b1sp, w2sp, b2sp,
             valo, idxo, sem):
    c = lax.axis_index("c")
    s = lax.axis_index("s")
    wid = s * 2 + c
    base = wid * _RPW

    cp = pltpu.async_copy(x_hbm.at[pl.ds(base * 10, _RPW * 10)], xv, sem)
    pltpu.sync_copy(w1_hbm, w1v)
    pltpu.sync_copy(b1_hbm, b1v)
    pltpu.sync_copy(w2_hbm, w2v)
    pltpu.sync_copy(b2_hbm, b2v)

    iota = lax.iota(jnp.int32, 16)
    iotac = jnp.minimum(iota, 9)

    for t in range(20):
        w1row = _round_bf16(plsc.load_gather(w1v, [iotac + 10 * t]))
        for i in range(10):
            w1sp[pl.ds(16 * (10 * t + i), 16)] = _rsplat(w1row, i)
    b1a = plsc.load_gather(b1v, [iota])
    b1b = plsc.load_gather(b1v, [jnp.minimum(iota + 16, 19)])
    for t in range(20):
        b1t = _rsplat(b1a, t) if t < 16 else _rsplat(b1b, t - 16)
        b1sp[pl.ds(16 * t, 16)] = b1t
    for t in range(30):
        w2col = _round_bf16(plsc.load_gather(w2v, [iotac * 30 + t]))
        for j in range(10):
            w2sp[pl.ds(16 * (10 * t + j), 16)] = _rsplat(w2col, j)
    b2vec = plsc.load_gather(b2v, [iotac])
    for j in range(10):
        b2sp[pl.ds(16 * j, 16)] = _rsplat(b2vec, j)

    cp.wait()

    neg_inf = jnp.full((16,), float("-inf"), dtype=jnp.float32)
    ten = _splat_i(10)
    iota10 = iota * 10
    iota3 = iota * 3

    def body(it, carry):
        g0 = it * _UNROLL
        riota10 = [iota10 + (g0 + u) * 160 for u in range(_UNROLL)]
        riota3 = [iota3 + (g0 + u) * 48 for u in range(_UNROLL)]
        xb = [[_round_bf16(plsc.load_gather(xv, [riota10[u] + i]))
               for i in range(10)] for u in range(_UNROLL)]
        acc = [[b2sp[pl.ds(16 * j, 16)] for j in range(10)]
               for u in range(_UNROLL)]
        for t in range(20):
            a = [b1sp[pl.ds(16 * t, 16)] for u in range(_UNROLL)]
            for i in range(10):
                w = w1sp[pl.ds(16 * (10 * t + i), 16)]
                for u in range(_UNROLL):
                    a[u] = a[u] + xb[u][i] * w
            x1b = [_round_bf16(a[u]) for u in range(_UNROLL)]
            for j in range(10):
                w = w2sp[pl.ds(16 * (10 * t + j), 16)]
                for u in range(_UNROLL):
                    acc[u][j] = acc[u][j] + x1b[u] * w
        for i in range(10):
            for j in range(10):
                w = w2sp[pl.ds(16 * (10 * (20 + i) + j), 16)]
                for u in range(_UNROLL):
                    acc[u][j] = acc[u][j] + xb[u][i] * w

        for u in range(_UNROLL):
            au = acc[u]
            for k in range(3):
                m = au[0]
                for j in range(1, 10):
                    m = jnp.maximum(m, au[j])
                idx = ten
                for j in range(9, -1, -1):
                    idx = jnp.where(au[j] == m, _splat_i(j), idx)
                plsc.store_scatter(valo, [riota3[u] + k], m)
                plsc.store_scatter(idxo, [riota3[u] + k], idx)
                if k < 2:
                    for j in range(10):
                        au[j] = jnp.where(idx == _splat_i(j), neg_inf, au[j])
        return carry

    lax.fori_loop(0, _ITERS, body, 0)

    pltpu.sync_copy(valo, val_hbm.at[pl.ds(base * 3, _RPW * 3)])
    pltpu.sync_copy(idxo, idx_hbm.at[pl.ds(base * 3, _RPW * 3)])


def kernel(x, W1, b1, W2, b2):
    n = x.shape[0]
    mesh = plsc.VectorSubcoreMesh(
        core_axis_name="c", subcore_axis_name="s",
        num_cores=2, num_subcores=16)
    run = pl.kernel(
        _sc_body,
        out_type=[
            jax.ShapeDtypeStruct((n * 3,), jnp.float32),
            jax.ShapeDtypeStruct((n * 3,), jnp.int32),
        ],
        mesh=mesh,
        compiler_params=pltpu.CompilerParams(needs_layout_passes=False),
        scratch_types=[
            pltpu.VMEM((_RPW * 10,), jnp.float32),
            pltpu.VMEM((200,), jnp.float32),
            pltpu.VMEM((20,), jnp.float32),
            pltpu.VMEM((300,), jnp.float32),
            pltpu.VMEM((10,), jnp.float32),
            pltpu.VMEM((3200,), jnp.float32),
            pltpu.VMEM((320,), jnp.float32),
            pltpu.VMEM((4800,), jnp.float32),
            pltpu.VMEM((160,), jnp.float32),
            pltpu.VMEM((_RPW * 3,), jnp.float32),
            pltpu.VMEM((_RPW * 3,), jnp.int32),
            pltpu.SemaphoreType.DMA,
        ],
    )
    vals, idxs = run(x.reshape(-1), W1.reshape(-1), b1, W2.reshape(-1), b2)
    return vals.reshape(n, 3), idxs.reshape(n, 3)


# speedup vs baseline: 1.1161x; 1.1161x over previous
import jax
import jax.numpy as jnp
from jax import lax
from jax.experimental import pallas as pl
from jax.experimental.pallas import tpu as pltpu
from jax.experimental.pallas import tpu_sc as plsc

_NW = 32
_N = 16384
_RPW = _N // _NW
_UNROLL = 1
_ITERS = _RPW // (16 * _UNROLL)


def _splat_i(v):
    return jnp.full((16,), v, dtype=jnp.int32)


def _rsplat(vec, k):
    return jnp.take_along_axis(vec, _splat_i(k), axis=0)


def _round_bf16(v):
    u = plsc.bitcast(v, jnp.int32)
    u = (u + jnp.int32(0x8000)) & jnp.int32(-0x10000)
    return plsc.bitcast(u, jnp.float32)


def _sc_body(x_hbm, w1_hbm, b1_hbm, w2_hbm, b2_hbm, val_hbm, idx_hbm,
             xv, w1v, b1v, w2v, b2v, w1sp, b1sp, w2sp, b2sp,
             valo, idxo, sem):
    c = lax.axis_index("c")
    s = lax.axis_index("s")
    wid = s * 2 + c
    base = wid * _RPW

    cp = pltpu.async_copy(x_hbm.at[pl.ds(base * 10, _RPW * 10)], xv, sem)
    pltpu.sync_copy(w1_hbm, w1v)
    pltpu.sync_copy(b1_hbm, b1v)
    pltpu.sync_copy(w2_hbm, w2v)
    pltpu.sync_copy(b2_hbm, b2v)

    iota = lax.iota(jnp.int32, 16)
    iotac = jnp.minimum(iota, 9)

    for t in range(20):
        w1row = _round_bf16(plsc.load_gather(w1v, [iotac + 10 * t]))
        for i in range(10):
            w1sp[pl.ds(16 * (10 * t + i), 16)] = _rsplat(w1row, i)
    b1a = plsc.load_gather(b1v, [iota])
    b1b = plsc.load_gather(b1v, [jnp.minimum(iota + 16, 19)])
    for t in range(20):
        b1t = _rsplat(b1a, t) if t < 16 else _rsplat(b1b, t - 16)
        b1sp[pl.ds(16 * t, 16)] = b1t
    for t in range(30):
        w2col = _round_bf16(plsc.load_gather(w2v, [iotac * 30 + t]))
        for j in range(10):
            w2sp[pl.ds(16 * (10 * t + j), 16)] = _rsplat(w2col, j)
    b2vec = plsc.load_gather(b2v, [iotac])
    for j in range(10):
        b2sp[pl.ds(16 * j, 16)] = _rsplat(b2vec, j)

    cp.wait()

    neg_inf = jnp.full((16,), float("-inf"), dtype=jnp.float32)
    ten = _splat_i(10)
    iota10 = iota * 10
    iota3 = iota * 3

    def body(it, carry):
        g0 = it * _UNROLL
        riota10 = [iota10 + (g0 + u) * 160 for u in range(_UNROLL)]
        riota3 = [iota3 + (g0 + u) * 48 for u in range(_UNROLL)]
        xb = [[_round_bf16(plsc.load_gather(xv, [riota10[u] + i]))
               for i in range(10)] for u in range(_UNROLL)]
        acc = [[b2sp[pl.ds(16 * j, 16)] for j in range(10)]
               for u in range(_UNROLL)]
        for t in range(20):
            a = [b1sp[pl.ds(16 * t, 16)] for u in range(_UNROLL)]
            for i in range(10):
                w = w1sp[pl.ds(16 * (10 * t + i), 16)]
                for u in range(_UNROLL):
                    a[u] = a[u] + xb[u][i] * w
            x1b = [_round_bf16(a[u]) for u in range(_UNROLL)]
            for j in range(10):
                w = w2sp[pl.ds(16 * (10 * t + j), 16)]
                for u in range(_UNROLL):
                    acc[u][j] = acc[u][j] + x1b[u] * w
        for i in range(10):
            for j in range(10):
                w = w2sp[pl.ds(16 * (10 * (20 + i) + j), 16)]
                for u in range(_UNROLL):
                    acc[u][j] = acc[u][j] + xb[u][i] * w

        for u in range(_UNROLL):
            au = acc[u]
            for k in range(3):
                m = au[0]
                for j in range(1, 10):
                    m = jnp.maximum(m, au[j])
                idx = ten
                for j in range(9, -1, -1):
                    idx = jnp.where(au[j] == m, _splat_i(j), idx)
                plsc.store_scatter(valo, [riota3[u] + k], m)
                plsc.store_scatter(idxo, [riota3[u] + k], idx)
                if k < 2:
                    for j in range(10):
                        au[j] = jnp.where(idx == _splat_i(j), neg_inf, au[j])
        return carry

    lax.fori_loop(0, _ITERS, body, 0)

    pltpu.sync_copy(valo, val_hbm.at[pl.ds(base * 3, _RPW * 3)])
    pltpu.sync_copy(idxo, idx_hbm.at[pl.ds(base * 3, _RPW * 3)])


def kernel(x, W1, b1, W2, b2):
    n = x.shape[0]
    mesh = plsc.VectorSubcoreMesh(
        core_axis_name="c", subcore_axis_name="s",
        num_cores=2, num_subcores=16)
    run = pl.kernel(
        _sc_body,
        out_type=[
            jax.ShapeDtypeStruct((n * 3,), jnp.float32),
            jax.ShapeDtypeStruct((n * 3,), jnp.int32),
        ],
        mesh=mesh,
        compiler_params=pltpu.CompilerParams(needs_layout_passes=False),
        scratch_types=[
            pltpu.VMEM((_RPW * 10,), jnp.float32),
            pltpu.VMEM((200,), jnp.float32),
            pltpu.VMEM((20,), jnp.float32),
            pltpu.VMEM((300,), jnp.float32),
            pltpu.VMEM((10,), jnp.float32),
            pltpu.VMEM((3200,), jnp.float32),
            pltpu.VMEM((320,), jnp.float32),
            pltpu.VMEM((4800,), jnp.float32),
            pltpu.VMEM((160,), jnp.float32),
            pltpu.VMEM((_RPW * 3,), jnp.float32),
            pltpu.VMEM((_RPW * 3,), jnp.int32),
            pltpu.SemaphoreType.DMA,
        ],
    )
    vals, idxs = run(x.reshape(-1), W1.reshape(-1), b1, W2.reshape(-1), b2)
    return vals.reshape(n, 3), idxs.reshape(n, 3)


# speedup vs baseline: 1.3281x; 1.1900x over previous
import jax
import jax.numpy as jnp
from jax import lax
from jax.experimental import pallas as pl
from jax.experimental.pallas import tpu as pltpu
from jax.experimental.pallas import tpu_sc as plsc

_NW = 32
_N = 16384
_RPW = _N // _NW
_UNROLL = 2
_ITERS = _RPW // (16 * _UNROLL)


def _splat_i(v):
    return jnp.full((16,), v, dtype=jnp.int32)


def _rsplat(vec, k):
    return jnp.take_along_axis(vec, _splat_i(k), axis=0)


def _round_bf16(v):
    u = plsc.bitcast(v, jnp.int32)
    u = (u + jnp.int32(0x8000)) & jnp.int32(-0x10000)
    return plsc.bitcast(u, jnp.float32)


def _sc_body(x_hbm, w1_hbm, b1_hbm, w2_hbm, b2_hbm, val_hbm, idx_hbm,
             xv, w1v, b1v, w2v, b2v, w1sp, b1sp, w2sp, b2sp,
             valo, idxo, sem):
    c = lax.axis_index("c")
    s = lax.axis_index("s")
    wid = s * 2 + c
    base = wid * _RPW

    cp = pltpu.async_copy(x_hbm.at[pl.ds(base * 10, _RPW * 10)], xv, sem)
    pltpu.sync_copy(w1_hbm, w1v)
    pltpu.sync_copy(b1_hbm, b1v)
    pltpu.sync_copy(w2_hbm, w2v)
    pltpu.sync_copy(b2_hbm, b2v)

    iota = lax.iota(jnp.int32, 16)
    iotac = jnp.minimum(iota, 9)

    for t in range(20):
        w1row = _round_bf16(plsc.load_gather(w1v, [iotac + 10 * t]))
        for i in range(10):
            w1sp[pl.ds(16 * (10 * t + i), 16)] = _rsplat(w1row, i)
    b1a = plsc.load_gather(b1v, [iota])
    b1b = plsc.load_gather(b1v, [jnp.minimum(iota + 16, 19)])
    for t in range(20):
        b1t = _rsplat(b1a, t) if t < 16 else _rsplat(b1b, t - 16)
        b1sp[pl.ds(16 * t, 16)] = b1t
    for t in range(30):
        w2col = _round_bf16(plsc.load_gather(w2v, [iotac * 30 + t]))
        for j in range(10):
            w2sp[pl.ds(16 * (10 * t + j), 16)] = _rsplat(w2col, j)
    b2vec = plsc.load_gather(b2v, [iotac])
    for j in range(10):
        b2sp[pl.ds(16 * j, 16)] = _rsplat(b2vec, j)

    cp.wait()

    neg_inf = jnp.full((16,), float("-inf"), dtype=jnp.float32)
    ten = _splat_i(10)
    iota10 = iota * 10
    iota3 = iota * 3

    def body(it, carry):
        g0 = it * _UNROLL
        riota10 = [iota10 + (g0 + u) * 160 for u in range(_UNROLL)]
        riota3 = [iota3 + (g0 + u) * 48 for u in range(_UNROLL)]
        xb = [[_round_bf16(plsc.load_gather(xv, [riota10[u] + i]))
               for i in range(10)] for u in range(_UNROLL)]
        b2j = [b2sp[pl.ds(16 * j, 16)] for j in range(10)]
        acc = [[b2j[j] for j in range(10)] for u in range(_UNROLL)]
        for t in range(20):
            b1t = b1sp[pl.ds(16 * t, 16)]
            a = [b1t for u in range(_UNROLL)]
            for i in range(10):
                w = w1sp[pl.ds(16 * (10 * t + i), 16)]
                for u in range(_UNROLL):
                    a[u] = a[u] + xb[u][i] * w
            x1b = [_round_bf16(a[u]) for u in range(_UNROLL)]
            for j in range(10):
                w = w2sp[pl.ds(16 * (10 * t + j), 16)]
                for u in range(_UNROLL):
                    acc[u][j] = acc[u][j] + x1b[u] * w
        for i in range(10):
            for j in range(10):
                w = w2sp[pl.ds(16 * (10 * (20 + i) + j), 16)]
                for u in range(_UNROLL):
                    acc[u][j] = acc[u][j] + xb[u][i] * w

        for u in range(_UNROLL):
            au = acc[u]
            for k in range(3):
                m = au[0]
                for j in range(1, 10):
                    m = jnp.maximum(m, au[j])
                idx = ten
                for j in range(9, -1, -1):
                    idx = jnp.where(au[j] == m, _splat_i(j), idx)
                plsc.store_scatter(valo, [riota3[u] + k], m)
                plsc.store_scatter(idxo, [riota3[u] + k], idx)
                if k < 2:
                    for j in range(10):
                        au[j] = jnp.where(idx == _splat_i(j), neg_inf, au[j])
        return carry

    lax.fori_loop(0, _ITERS, body, 0)

    pltpu.sync_copy(valo, val_hbm.at[pl.ds(base * 3, _RPW * 3)])
    pltpu.sync_copy(idxo, idx_hbm.at[pl.ds(base * 3, _RPW * 3)])


def kernel(x, W1, b1, W2, b2):
    n = x.shape[0]
    mesh = plsc.VectorSubcoreMesh(
        core_axis_name="c", subcore_axis_name="s",
        num_cores=2, num_subcores=16)
    run = pl.kernel(
        _sc_body,
        out_type=[
            jax.ShapeDtypeStruct((n * 3,), jnp.float32),
            jax.ShapeDtypeStruct((n * 3,), jnp.int32),
        ],
        mesh=mesh,
        compiler_params=pltpu.CompilerParams(needs_layout_passes=False),
        scratch_types=[
            pltpu.VMEM((_RPW * 10,), jnp.float32),
            pltpu.VMEM((200,), jnp.float32),
            pltpu.VMEM((20,), jnp.float32),
            pltpu.VMEM((300,), jnp.float32),
            pltpu.VMEM((10,), jnp.float32),
            pltpu.VMEM((3200,), jnp.float32),
            pltpu.VMEM((320,), jnp.float32),
            pltpu.VMEM((4800,), jnp.float32),
            pltpu.VMEM((160,), jnp.float32),
            pltpu.VMEM((_RPW * 3,), jnp.float32),
            pltpu.VMEM((_RPW * 3,), jnp.int32),
            pltpu.SemaphoreType.DMA,
        ],
    )
    vals, idxs = run(x.reshape(-1), W1.reshape(-1), b1, W2.reshape(-1), b2)
    return vals.reshape(n, 3), idxs.reshape(n, 3)


# speedup vs baseline: 1.6123x; 1.2140x over previous
import jax
import jax.numpy as jnp
from jax import lax
from jax.experimental import pallas as pl
from jax.experimental.pallas import tpu as pltpu
from jax.experimental.pallas import tpu_sc as plsc

_NW = 32
_SC_N = 2048
_RPW = _SC_N // _NW
_UNROLL = 2
_ITERS = _RPW // (16 * _UNROLL)

_TC_BLOCK = 2048
_NEG_INF = float("-inf")


def _splat_i(v):
    return jnp.full((16,), v, dtype=jnp.int32)


def _rsplat(vec, k):
    return jnp.take_along_axis(vec, _splat_i(k), axis=0)


def _round_bf16(v):
    u = plsc.bitcast(v, jnp.int32)
    u = (u + jnp.int32(0x8000)) & jnp.int32(-0x10000)
    return plsc.bitcast(u, jnp.float32)


def _sc_body(x_hbm, w1_hbm, b1_hbm, w2_hbm, b2_hbm, val_hbm, idx_hbm,
             xv, w1v, b1v, w2v, b2v, w1sp, b1sp, w2sp, b2sp,
             valo, idxo, sem):
    c = lax.axis_index("c")
    s = lax.axis_index("s")
    wid = s * 2 + c
    base = wid * _RPW

    cp = pltpu.async_copy(x_hbm.at[pl.ds(base * 10, _RPW * 10)], xv, sem)
    pltpu.sync_copy(w1_hbm, w1v)
    pltpu.sync_copy(b1_hbm, b1v)
    pltpu.sync_copy(w2_hbm, w2v)
    pltpu.sync_copy(b2_hbm, b2v)

    iota = lax.iota(jnp.int32, 16)
    iotac = jnp.minimum(iota, 9)

    for t in range(20):
        w1row = _round_bf16(plsc.load_gather(w1v, [iotac + 10 * t]))
        for i in range(10):
            w1sp[pl.ds(16 * (10 * t + i), 16)] = _rsplat(w1row, i)
    b1a = plsc.load_gather(b1v, [iota])
    b1b = plsc.load_gather(b1v, [jnp.minimum(iota + 16, 19)])
    for t in range(20):
        b1t = _rsplat(b1a, t) if t < 16 else _rsplat(b1b, t - 16)
        b1sp[pl.ds(16 * t, 16)] = b1t
    for t in range(30):
        w2col = _round_bf16(plsc.load_gather(w2v, [iotac * 30 + t]))
        for j in range(10):
            w2sp[pl.ds(16 * (10 * t + j), 16)] = _rsplat(w2col, j)
    b2vec = plsc.load_gather(b2v, [iotac])
    for j in range(10):
        b2sp[pl.ds(16 * j, 16)] = _rsplat(b2vec, j)

    cp.wait()

    neg_inf = jnp.full((16,), float("-inf"), dtype=jnp.float32)
    ten = _splat_i(10)
    iota10 = iota * 10
    iota3 = iota * 3

    def body(it, carry):
        g0 = it * _UNROLL
        riota10 = [iota10 + (g0 + u) * 160 for u in range(_UNROLL)]
        riota3 = [iota3 + (g0 + u) * 48 for u in range(_UNROLL)]
        xb = [[_round_bf16(plsc.load_gather(xv, [riota10[u] + i]))
               for i in range(10)] for u in range(_UNROLL)]
        b2j = [b2sp[pl.ds(16 * j, 16)] for j in range(10)]
        acc = [[b2j[j] for j in range(10)] for u in range(_UNROLL)]
        for t in range(20):
            b1t = b1sp[pl.ds(16 * t, 16)]
            a = [b1t for u in range(_UNROLL)]
            for i in range(10):
                w = w1sp[pl.ds(16 * (10 * t + i), 16)]
                for u in range(_UNROLL):
                    a[u] = a[u] + xb[u][i] * w
            x1b = [_round_bf16(a[u]) for u in range(_UNROLL)]
            for j in range(10):
                w = w2sp[pl.ds(16 * (10 * t + j), 16)]
                for u in range(_UNROLL):
                    acc[u][j] = acc[u][j] + x1b[u] * w
        for i in range(10):
            for j in range(10):
                w = w2sp[pl.ds(16 * (10 * (20 + i) + j), 16)]
                for u in range(_UNROLL):
                    acc[u][j] = acc[u][j] + xb[u][i] * w

        for u in range(_UNROLL):
            au = acc[u]
            for k in range(3):
                m = au[0]
                for j in range(1, 10):
                    m = jnp.maximum(m, au[j])
                idx = ten
                for j in range(9, -1, -1):
                    idx = jnp.where(au[j] == m, _splat_i(j), idx)
                plsc.store_scatter(valo, [riota3[u] + k], m)
                plsc.store_scatter(idxo, [riota3[u] + k], idx)
                if k < 2:
                    for j in range(10):
                        au[j] = jnp.where(idx == _splat_i(j), neg_inf, au[j])
        return carry

    lax.fori_loop(0, _ITERS, body, 0)

    pltpu.sync_copy(valo, val_hbm.at[pl.ds(base * 3, _RPW * 3)])
    pltpu.sync_copy(idxo, idx_hbm.at[pl.ds(base * 3, _RPW * 3)])


def _tc_body(x_ref, w1_ref, b1_ref, w2_ref, b2_ref, val_ref, idx_ref):
    x = x_ref[...]
    w1 = w1_ref[...]
    w2 = w2_ref[...]
    dn = (((1,), (1,)), ((), ()))
    x1 = lax.dot_general(x, w1, dn, preferred_element_type=jnp.float32)
    x1 = x1 + b1_ref[...]
    x2 = jnp.concatenate([x1, x], axis=1)
    x3 = lax.dot_general(x2, w2, dn, preferred_element_type=jnp.float32)
    x3 = x3 + b2_ref[...]

    iota = lax.broadcasted_iota(jnp.int32, x3.shape, 1)
    vals = x3
    out_v = []
    out_i = []
    for _ in range(3):
        m = jnp.max(vals, axis=1, keepdims=True)
        idx = jnp.min(jnp.where(vals == m, iota, 10), axis=1, keepdims=True)
        out_v.append(m)
        out_i.append(idx)
        vals = jnp.where(iota == idx, _NEG_INF, vals)
    val_ref[...] = jnp.concatenate(out_v, axis=1)
    idx_ref[...] = jnp.concatenate(out_i, axis=1)


def kernel(x, W1, b1, W2, b2):
    n = x.shape[0]
    xs, xt = x[:_SC_N], x[_SC_N:]

    mesh = plsc.VectorSubcoreMesh(
        core_axis_name="c", subcore_axis_name="s",
        num_cores=2, num_subcores=16)
    run = pl.kernel(
        _sc_body,
        out_type=[
            jax.ShapeDtypeStruct((_SC_N * 3,), jnp.float32),
            jax.ShapeDtypeStruct((_SC_N * 3,), jnp.int32),
        ],
        mesh=mesh,
        compiler_params=pltpu.CompilerParams(needs_layout_passes=False),
        scratch_types=[
            pltpu.VMEM((_RPW * 10,), jnp.float32),
            pltpu.VMEM((200,), jnp.float32),
            pltpu.VMEM((20,), jnp.float32),
            pltpu.VMEM((300,), jnp.float32),
            pltpu.VMEM((10,), jnp.float32),
            pltpu.VMEM((3200,), jnp.float32),
            pltpu.VMEM((320,), jnp.float32),
            pltpu.VMEM((4800,), jnp.float32),
            pltpu.VMEM((160,), jnp.float32),
            pltpu.VMEM((_RPW * 3,), jnp.float32),
            pltpu.VMEM((_RPW * 3,), jnp.int32),
            pltpu.SemaphoreType.DMA,
        ],
    )
    sc_vals, sc_idxs = run(xs.reshape(-1), W1.reshape(-1), b1,
                           W2.reshape(-1), b2)

    nt = n - _SC_N
    tc_vals, tc_idxs = pl.pallas_call(
        _tc_body,
        grid=(nt // _TC_BLOCK,),
        in_specs=[
            pl.BlockSpec((_TC_BLOCK, 10), lambda i: (i, 0)),
            pl.BlockSpec((20, 10), lambda i: (0, 0)),
            pl.BlockSpec((1, 20), lambda i: (0, 0)),
            pl.BlockSpec((10, 30), lambda i: (0, 0)),
            pl.BlockSpec((1, 10), lambda i: (0, 0)),
        ],
        out_specs=[
            pl.BlockSpec((_TC_BLOCK, 3), lambda i: (i, 0)),
            pl.BlockSpec((_TC_BLOCK, 3), lambda i: (i, 0)),
        ],
        out_shape=[
            jax.ShapeDtypeStruct((nt, 3), jnp.float32),
            jax.ShapeDtypeStruct((nt, 3), jnp.int32),
        ],
    )(xt, W1, b1.reshape(1, 20), W2, b2.reshape(1, 10))

    vals = jnp.concatenate([sc_vals.reshape(_SC_N, 3), tc_vals], axis=0)
    idxs = jnp.concatenate([sc_idxs.reshape(_SC_N, 3), tc_idxs], axis=0)
    return vals, idxs
